# baseline (speedup 1.0000x reference)
import functools

import jax
import jax.numpy as jnp
from jax import lax
from jax.experimental import pallas as pl
from jax.experimental.pallas import tpu as pltpu
from jax.experimental.pallas import tpu_sc as plsc

N_USER = 5000
N_ITEM = 5000
N = N_USER + N_ITEM
HID = 128
NB = 3
E = 320000

NC = 2
NT = 16
NW = NC * NT
LANES = 16
DPT = HID // NW
ES = 4096
NSLAB = 80
E_PAD = NSLAB * ES
NGRP = ES // LANES


def _sc_spmm(emb_t, col, row, val):
    mesh = plsc.VectorSubcoreMesh(core_axis_name="c", subcore_axis_name="s",
                                  num_cores=NC, num_subcores=NT)

    @functools.partial(
        pl.kernel,
        out_type=jax.ShapeDtypeStruct((NB, NW, N * DPT), jnp.float32),
        mesh=mesh,
        scratch_types=[
            pltpu.VMEM((N * DPT,), jnp.float32),
            pltpu.VMEM((N * DPT,), jnp.float32),
            pltpu.VMEM((ES,), jnp.int32),
            pltpu.VMEM((ES,), jnp.int32),
            pltpu.VMEM((ES,), jnp.float32),
            pltpu.VMEM((ES,), jnp.int32),
            pltpu.VMEM((ES,), jnp.int32),
            pltpu.VMEM((ES,), jnp.float32),
            pltpu.SemaphoreType.DMA,
            pltpu.SemaphoreType.DMA,
            pltpu.SemaphoreType.DMA,
            pltpu.SemaphoreType.DMA,
            pltpu.SemaphoreType.DMA,
            pltpu.SemaphoreType.DMA,
        ],
        compiler_params=pltpu.CompilerParams(needs_layout_passes=False),
    )
    def spmm_kernel(emb_hbm, col_hbm, row_hbm, val_hbm, out_hbm,
                    tbl, acc, ecol0, erow0, eval0, ecol1, erow1, eval1,
                    sa0, sb0, sc0, sa1, sb1, sc1):
        c = lax.axis_index("c")
        s = lax.axis_index("s")
        t = c * NT + s
        bufs = ((ecol0, erow0, eval0, sa0, sb0, sc0),
                (ecol1, erow1, eval1, sa1, sb1, sc1))

        def start_slab(b, sl, which):
            ecol, erow, eval_, sa, sb, sc_ = bufs[which]
            return (pltpu.async_copy(col_hbm.at[b, sl, 0], ecol, sa),
                    pltpu.async_copy(row_hbm.at[b, sl, 0], erow, sb),
                    pltpu.async_copy(val_hbm.at[b, sl, 0], eval_, sc_))

        def process_slab(from_buf, to_buf, which):
            ecol, erow, eval_, _, _, _ = bufs[which]

            @plsc.parallel_loop(0, NGRP, 1, unroll=8)
            def group_body(g):
                base = g * LANES
                colv = ecol[pl.ds(base, LANES)]
                rowv = erow[pl.ds(base, LANES)]
                valv = eval_[pl.ds(base, LANES)]
                gvs = [plsc.load_gather(from_buf.at[pl.ds(d * N, N)], [colv])
                       for d in range(DPT)]
                for d in range(DPT):
                    plsc.addupdate_scatter(to_buf.at[pl.ds(d * N, N)], [rowv],
                                           gvs[d] * valv)

        def hop(b, from_buf, to_buf):
            for cp in start_slab(b, 0, 0):
                cp.wait()
            last = NSLAB - 1

            def pair_body(p, carry):
                sl = p * 2
                nxt1 = start_slab(b, lax.min(sl + 1, last), 1)
                process_slab(from_buf, to_buf, 0)
                for cp in nxt1:
                    cp.wait()
                nxt0 = start_slab(b, lax.min(sl + 2, last), 0)
                process_slab(from_buf, to_buf, 1)
                for cp in nxt0:
                    cp.wait()
                return carry

            lax.fori_loop(0, NSLAB // 2, pair_body, 0)

        for b in range(NB):
            pltpu.sync_copy(emb_hbm.at[t], tbl)
            pltpu.sync_copy(emb_hbm.at[t], acc)
            hop(b, tbl, acc)
            pltpu.sync_copy(emb_hbm.at[t], tbl)
            hop(b, acc, tbl)
            pltpu.sync_copy(tbl, out_hbm.at[b, t])

    return spmm_kernel(emb_t, col, row, val)


def _tc_dense(tem, uHyper, iHyper, u_w, i_w):
    def body(tem_ref, uh_ref, ih_ref, uw_ref, iw_ref,
             ue_ref, ie_ref, uE_ref, iE_ref, su_ref, si_ref):
        b = pl.program_id(0)
        uw = uw_ref[...]
        iw = iw_ref[...]
        uh = uh_ref[...]
        ih = ih_ref[...]
        hu = lax.dot_general(uh, uh, (((0,), (0,)), ((), ())),
                             preferred_element_type=jnp.float32)
        hi = lax.dot_general(ih, ih, (((0,), (0,)), ((), ())),
                             preferred_element_type=jnp.float32)
        tu = tem_ref[0, :N_USER, :]
        ti = tem_ref[0, N_USER:, :]
        gu = lax.dot_general(tu, tu, (((0,), (0,)), ((), ())),
                             preferred_element_type=jnp.float32)
        gi = lax.dot_general(ti, ti, (((0,), (0,)), ((), ())),
                             preferred_element_type=jnp.float32)
        ub = jnp.dot(jnp.dot(tu, hu, preferred_element_type=jnp.float32), gu,
                     preferred_element_type=jnp.float32)
        ib = jnp.dot(jnp.dot(ti, hi, preferred_element_type=jnp.float32), gi,
                     preferred_element_type=jnp.float32)
        uE_ref[0] = jnp.maximum(
            jnp.dot(ub, uw, preferred_element_type=jnp.float32), 0.0)
        iE_ref[0] = jnp.maximum(
            jnp.dot(ib, iw, preferred_element_type=jnp.float32), 0.0)

        @pl.when(b == 0)
        def _():
            su_ref[...] = ub
            si_ref[...] = ib

        @pl.when(b > 0)
        def _():
            su_ref[...] += ub
            si_ref[...] += ib

        @pl.when(b == NB - 1)
        def _():
            ue_ref[...] = jnp.maximum(
                jnp.dot(su_ref[...] / NB, uw,
                        preferred_element_type=jnp.float32), 0.0)
            ie_ref[...] = jnp.maximum(
                jnp.dot(si_ref[...] / NB, iw,
                        preferred_element_type=jnp.float32), 0.0)

    full128 = pl.BlockSpec((HID, HID), lambda b: (0, 0))
    return pl.pallas_call(
        body,
        grid=(NB,),
        in_specs=[
            pl.BlockSpec((1, N, HID), lambda b: (b, 0, 0)),
            full128, full128, full128, full128,
        ],
        out_specs=[
            pl.BlockSpec((N_USER, HID), lambda b: (0, 0)),
            pl.BlockSpec((N_ITEM, HID), lambda b: (0, 0)),
            pl.BlockSpec((1, N_USER, HID), lambda b: (b, 0, 0)),
            pl.BlockSpec((1, N_ITEM, HID), lambda b: (b, 0, 0)),
        ],
        out_shape=[
            jax.ShapeDtypeStruct((N_USER, HID), jnp.float32),
            jax.ShapeDtypeStruct((N_ITEM, HID), jnp.float32),
            jax.ShapeDtypeStruct((NB, N_USER, HID), jnp.float32),
            jax.ShapeDtypeStruct((NB, N_ITEM, HID), jnp.float32),
        ],
        scratch_shapes=[
            pltpu.VMEM((N_USER, HID), jnp.float32),
            pltpu.VMEM((N_ITEM, HID), jnp.float32),
        ],
        compiler_params=pltpu.CompilerParams(
            dimension_semantics=("arbitrary",),
        ),
    )(tem, uHyper, iHyper, u_w, i_w)


def kernel(user_embedding, item_embedding, uEmbeds, iEmbeds, uHyper, iHyper,
           u_w, i_w, edge_val, edge_row, edge_col):
    embeds0 = jnp.concatenate([uEmbeds, iEmbeds], axis=0)
    emb_t = embeds0.reshape(N, NW, DPT).transpose(1, 2, 0).reshape(NW, DPT * N)
    pad = E_PAD - E
    eshape = (NB, NSLAB, 1, ES)
    col = jnp.pad(edge_col, ((0, 0), (0, pad))).reshape(eshape)
    row = jnp.pad(edge_row, ((0, 0), (0, pad))).reshape(eshape)
    val = jnp.pad(edge_val, ((0, 0), (0, pad))).reshape(eshape)
    tem_t = _sc_spmm(emb_t, col, row, val)
    tem = (tem_t.reshape(NB, NW, DPT, N).transpose(0, 3, 1, 2)
           .reshape(NB, N, HID))
    ue, ie, uE, iE = _tc_dense(tem, uHyper, iHyper, u_w, i_w)
    return (ue, ie, uE, iE)


# speedup vs baseline: 1.0639x; 1.0639x over previous
import functools

import jax
import jax.numpy as jnp
from jax import lax
from jax.experimental import pallas as pl
from jax.experimental.pallas import tpu as pltpu
from jax.experimental.pallas import tpu_sc as plsc

N_USER = 5000
N_ITEM = 5000
N = N_USER + N_ITEM
HID = 128
NB = 3
E = 320000

NC = 2
NT = 16
NW = NC * NT
LANES = 16
DPT = HID // NW
ES = 2048
NSLAB = 158
E_PAD = NSLAB * ES
NGRP = ES // LANES


def _sc_spmm(emb_t, col, row, val):
    mesh = plsc.VectorSubcoreMesh(core_axis_name="c", subcore_axis_name="s",
                                  num_cores=NC, num_subcores=NT)

    @functools.partial(
        pl.kernel,
        out_type=jax.ShapeDtypeStruct((NB, NW, N * DPT), jnp.float32),
        mesh=mesh,
        scratch_types=[
            pltpu.VMEM((N * DPT,), jnp.float32),
            pltpu.VMEM((N * DPT,), jnp.float32),
            pltpu.VMEM((ES,), jnp.int32),
            pltpu.VMEM((ES,), jnp.int32),
            pltpu.VMEM((ES,), jnp.float32),
            pltpu.VMEM((ES,), jnp.int32),
            pltpu.VMEM((ES,), jnp.int32),
            pltpu.VMEM((ES,), jnp.float32),
            pltpu.SemaphoreType.DMA,
            pltpu.SemaphoreType.DMA,
            pltpu.SemaphoreType.DMA,
            pltpu.SemaphoreType.DMA,
            pltpu.SemaphoreType.DMA,
            pltpu.SemaphoreType.DMA,
        ],
        compiler_params=pltpu.CompilerParams(needs_layout_passes=False),
    )
    def spmm_kernel(emb_hbm, col_hbm, row_hbm, val_hbm, out_hbm,
                    tbl, acc, ecol0, erow0, eval0, ecol1, erow1, eval1,
                    sa0, sb0, sc0, sa1, sb1, sc1):
        c = lax.axis_index("c")
        s = lax.axis_index("s")
        t = c * NT + s
        bufs = ((ecol0, erow0, eval0, sa0, sb0, sc0),
                (ecol1, erow1, eval1, sa1, sb1, sc1))

        def start_slab(b, sl, which):
            ecol, erow, eval_, sa, sb, sc_ = bufs[which]
            return (pltpu.async_copy(col_hbm.at[b, sl, 0], ecol, sa),
                    pltpu.async_copy(row_hbm.at[b, sl, 0], erow, sb),
                    pltpu.async_copy(val_hbm.at[b, sl, 0], eval_, sc_))

        def process_slab(from_buf, to_buf, which):
            ecol, erow, eval_, _, _, _ = bufs[which]

            @plsc.parallel_loop(0, NGRP, 1, unroll=8)
            def group_body(g):
                base = g * LANES
                colv = ecol[pl.ds(base, LANES)]
                rowv = erow[pl.ds(base, LANES)]
                valv = eval_[pl.ds(base, LANES)]
                gvs = [plsc.load_gather(from_buf.at[pl.ds(d * N, N)], [colv])
                       for d in range(DPT)]
                for d in range(DPT):
                    plsc.addupdate_scatter(to_buf.at[pl.ds(d * N, N)], [rowv],
                                           gvs[d] * valv)

        def hop(b, from_buf, to_buf):
            for cp in start_slab(b, 0, 0):
                cp.wait()
            last = NSLAB - 1

            def pair_body(p, carry):
                sl = p * 2
                nxt1 = start_slab(b, lax.min(sl + 1, last), 1)
                process_slab(from_buf, to_buf, 0)
                for cp in nxt1:
                    cp.wait()
                nxt0 = start_slab(b, lax.min(sl + 2, last), 0)
                process_slab(from_buf, to_buf, 1)
                for cp in nxt0:
                    cp.wait()
                return carry

            lax.fori_loop(0, NSLAB // 2, pair_body, 0)

        for b in range(NB):
            pltpu.sync_copy(emb_hbm.at[t], tbl)
            pltpu.sync_copy(emb_hbm.at[t], acc)
            hop(b, tbl, acc)
            pltpu.sync_copy(emb_hbm.at[t], tbl)
            hop(b, acc, tbl)
            pltpu.sync_copy(tbl, out_hbm.at[b, t])

    return spmm_kernel(emb_t, col, row, val)


def _tc_dense(tem, uHyper, iHyper, u_w, i_w):
    def body(tem_ref, uh_ref, ih_ref, uw_ref, iw_ref,
             ue_ref, ie_ref, uE_ref, iE_ref, su_ref, si_ref):
        b = pl.program_id(0)
        uw = uw_ref[...]
        iw = iw_ref[...]
        uh = uh_ref[...]
        ih = ih_ref[...]
        hu = lax.dot_general(uh, uh, (((0,), (0,)), ((), ())),
                             preferred_element_type=jnp.float32)
        hi = lax.dot_general(ih, ih, (((0,), (0,)), ((), ())),
                             preferred_element_type=jnp.float32)
        tu = tem_ref[0, :N_USER, :]
        ti = tem_ref[0, N_USER:, :]
        gu = lax.dot_general(tu, tu, (((0,), (0,)), ((), ())),
                             preferred_element_type=jnp.float32)
        gi = lax.dot_general(ti, ti, (((0,), (0,)), ((), ())),
                             preferred_element_type=jnp.float32)
        ub = jnp.dot(jnp.dot(tu, hu, preferred_element_type=jnp.float32), gu,
                     preferred_element_type=jnp.float32)
        ib = jnp.dot(jnp.dot(ti, hi, preferred_element_type=jnp.float32), gi,
                     preferred_element_type=jnp.float32)
        uE_ref[0] = jnp.maximum(
            jnp.dot(ub, uw, preferred_element_type=jnp.float32), 0.0)
        iE_ref[0] = jnp.maximum(
            jnp.dot(ib, iw, preferred_element_type=jnp.float32), 0.0)

        @pl.when(b == 0)
        def _():
            su_ref[...] = ub
            si_ref[...] = ib

        @pl.when(b > 0)
        def _():
            su_ref[...] += ub
            si_ref[...] += ib

        @pl.when(b == NB - 1)
        def _():
            ue_ref[...] = jnp.maximum(
                jnp.dot(su_ref[...] / NB, uw,
                        preferred_element_type=jnp.float32), 0.0)
            ie_ref[...] = jnp.maximum(
                jnp.dot(si_ref[...] / NB, iw,
                        preferred_element_type=jnp.float32), 0.0)

    full128 = pl.BlockSpec((HID, HID), lambda b: (0, 0))
    return pl.pallas_call(
        body,
        grid=(NB,),
        in_specs=[
            pl.BlockSpec((1, N, HID), lambda b: (b, 0, 0)),
            full128, full128, full128, full128,
        ],
        out_specs=[
            pl.BlockSpec((N_USER, HID), lambda b: (0, 0)),
            pl.BlockSpec((N_ITEM, HID), lambda b: (0, 0)),
            pl.BlockSpec((1, N_USER, HID), lambda b: (b, 0, 0)),
            pl.BlockSpec((1, N_ITEM, HID), lambda b: (b, 0, 0)),
        ],
        out_shape=[
            jax.ShapeDtypeStruct((N_USER, HID), jnp.float32),
            jax.ShapeDtypeStruct((N_ITEM, HID), jnp.float32),
            jax.ShapeDtypeStruct((NB, N_USER, HID), jnp.float32),
            jax.ShapeDtypeStruct((NB, N_ITEM, HID), jnp.float32),
        ],
        scratch_shapes=[
            pltpu.VMEM((N_USER, HID), jnp.float32),
            pltpu.VMEM((N_ITEM, HID), jnp.float32),
        ],
        compiler_params=pltpu.CompilerParams(
            dimension_semantics=("arbitrary",),
        ),
    )(tem, uHyper, iHyper, u_w, i_w)


def kernel(user_embedding, item_embedding, uEmbeds, iEmbeds, uHyper, iHyper,
           u_w, i_w, edge_val, edge_row, edge_col):
    embeds0 = jnp.concatenate([uEmbeds, iEmbeds], axis=0)
    emb_t = embeds0.reshape(N, NW, DPT).transpose(1, 2, 0).reshape(NW, DPT * N)
    pad = E_PAD - E
    eshape = (NB, NSLAB, 1, ES)
    col = jnp.pad(edge_col, ((0, 0), (0, pad))).reshape(eshape)
    row = jnp.pad(edge_row, ((0, 0), (0, pad))).reshape(eshape)
    val = jnp.pad(edge_val, ((0, 0), (0, pad))).reshape(eshape)
    tem_t = _sc_spmm(emb_t, col, row, val)
    tem = (tem_t.reshape(NB, NW, DPT, N).transpose(0, 3, 1, 2)
           .reshape(NB, N, HID))
    ue, ie, uE, iE = _tc_dense(tem, uHyper, iHyper, u_w, i_w)
    return (ue, ie, uE, iE)
